# Initial kernel scaffold; baseline (speedup 1.0000x reference)
import jax
import jax.numpy as jnp
from jax.experimental import pallas as pl


def kernel(t, pe):
    raise NotImplementedError("write your pallas kernel here")


# speedup vs baseline: 1.4777x; 1.4777x over previous
import functools

import jax
import jax.numpy as jnp
from jax import lax
from jax.experimental import pallas as pl
from jax.experimental.pallas import tpu as pltpu
from jax.experimental.pallas import tpu_sc as plsc


@functools.lru_cache(maxsize=None)
def _make_pe_gather(B, V, D):
    info = plsc.get_sparse_core_info()
    NC, NS, L = info.num_cores, info.num_subcores, info.num_lanes
    NW = NC * NS
    assert B % NW == 0 and D % L == 0
    b_per_w = B // NW
    CH = 128
    NCH = b_per_w // CH
    mesh = plsc.VectorSubcoreMesh(core_axis_name="c", subcore_axis_name="s")

    @functools.partial(
        pl.kernel,
        mesh=mesh,
        out_type=jax.ShapeDtypeStruct((B, D), jnp.float32),
        scratch_types=[
            pltpu.VMEM((b_per_w,), jnp.float32),
            pltpu.VMEM((NCH, CH), jnp.int32),
            pltpu.VMEM((CH, D), jnp.float32),
            pltpu.SemaphoreType.DMA,
        ],
    )
    def k(t_hbm, pe_hbm, out_hbm, t_v, idx_v, rows_v, sem):
        wid = lax.axis_index("s") * NC + lax.axis_index("c")
        base = wid * b_per_w
        pltpu.sync_copy(t_hbm.at[pl.ds(base, b_per_w)], t_v)
        scale = jnp.float32(V - 1)
        for c in range(NCH):
            for j in range(CH // L):
                tv = t_v[pl.ds(c * CH + j * L, L)]
                iv = jnp.clip((tv * scale).astype(jnp.int32), 0, V - 1)
                idx_v[c, pl.ds(j * L, L)] = iv
        for c in range(NCH):
            pltpu.async_copy(pe_hbm.at[idx_v.at[c]], rows_v, sem).wait()
            pltpu.sync_copy(rows_v, out_hbm.at[pl.ds(base + c * CH, CH)])

    return k


def kernel(t, pe):
    B, = t.shape
    V, D = pe.shape
    return _make_pe_gather(B, V, D)(t, pe)


# speedup vs baseline: 1.5210x; 1.0293x over previous
import functools

import jax
import jax.numpy as jnp
from jax import lax
from jax.experimental import pallas as pl
from jax.experimental.pallas import tpu as pltpu
from jax.experimental.pallas import tpu_sc as plsc


@functools.lru_cache(maxsize=None)
def _make_pe_gather(B, V, D):
    info = plsc.get_sparse_core_info()
    NC, NS, L = info.num_cores, info.num_subcores, info.num_lanes
    NW = NC * NS
    assert B % NW == 0 and D % L == 0
    b_per_w = B // NW
    CH = 64
    NCH = b_per_w // CH
    mesh = plsc.VectorSubcoreMesh(core_axis_name="c", subcore_axis_name="s")

    @functools.partial(
        pl.kernel,
        mesh=mesh,
        out_type=jax.ShapeDtypeStruct((B, D), jnp.float32),
        scratch_types=[
            pltpu.VMEM((b_per_w,), jnp.float32),
            pltpu.VMEM((NCH, CH), jnp.int32),
            pltpu.VMEM((2, CH, D), jnp.float32),
            pltpu.SemaphoreType.DMA,
            pltpu.SemaphoreType.DMA,
            pltpu.SemaphoreType.DMA,
            pltpu.SemaphoreType.DMA,
        ],
    )
    def k(t_hbm, pe_hbm, out_hbm, t_v, idx_v, rows_v, g0, g1, o0, o1):
        wid = lax.axis_index("s") * NC + lax.axis_index("c")
        base = wid * b_per_w
        pltpu.sync_copy(t_hbm.at[pl.ds(base, b_per_w)], t_v)
        scale = jnp.float32(V - 1)
        for c in range(NCH):
            for j in range(CH // L):
                tv = t_v[pl.ds(c * CH + j * L, L)]
                iv = jnp.clip((tv * scale).astype(jnp.int32), 0, V - 1)
                idx_v[c, pl.ds(j * L, L)] = iv

        gsem = (g0, g1)
        osem = (o0, o1)

        def gather(c):
            b = c & 1
            return pltpu.async_copy(pe_hbm.at[idx_v.at[c]], rows_v.at[b], gsem[b])

        def put(c):
            b = c & 1
            return pltpu.async_copy(
                rows_v.at[b], out_hbm.at[pl.ds(base + c * CH, CH)], osem[b])

        gpend = [None, None]
        opend = [None, None]
        gpend[0] = gather(0)
        for c in range(NCH):
            b = c & 1
            nb = (c + 1) & 1
            if c + 1 < NCH:
                if opend[nb] is not None:
                    opend[nb].wait()
                gpend[nb] = gather(c + 1)
            gpend[b].wait()
            opend[b] = put(c)
        for p in opend:
            if p is not None:
                p.wait()

    return k


def kernel(t, pe):
    B, = t.shape
    V, D = pe.shape
    return _make_pe_gather(B, V, D)(t, pe)


# speedup vs baseline: 1.5373x; 1.0107x over previous
import functools

import jax
import jax.numpy as jnp
from jax import lax
from jax.experimental import pallas as pl
from jax.experimental.pallas import tpu as pltpu
from jax.experimental.pallas import tpu_sc as plsc


@functools.lru_cache(maxsize=None)
def _make_pe_gather(B, V, D):
    info = plsc.get_sparse_core_info()
    NC, NS, L = info.num_cores, info.num_subcores, info.num_lanes
    NW = NC * NS
    assert B % NW == 0 and D % L == 0
    b_per_w = B // NW
    CH = 32
    NCH = b_per_w // CH
    NBUF = 4
    LEAD = NBUF - 1
    mesh = plsc.VectorSubcoreMesh(core_axis_name="c", subcore_axis_name="s")

    @functools.partial(
        pl.kernel,
        mesh=mesh,
        out_type=jax.ShapeDtypeStruct((B, D), jnp.float32),
        scratch_types=[
            pltpu.VMEM((b_per_w,), jnp.float32),
            pltpu.VMEM((NCH, CH), jnp.int32),
            pltpu.VMEM((NBUF, CH, D), jnp.float32),
        ]
        + [pltpu.SemaphoreType.DMA] * (2 * NBUF),
    )
    def k(t_hbm, pe_hbm, out_hbm, t_v, idx_v, rows_v, *sems):
        gsem = sems[:NBUF]
        osem = sems[NBUF:]
        wid = lax.axis_index("s") * NC + lax.axis_index("c")
        base = wid * b_per_w
        pltpu.sync_copy(t_hbm.at[pl.ds(base, b_per_w)], t_v)
        scale = jnp.float32(V - 1)

        def compute_idx(c):
            for j in range(CH // L):
                tv = t_v[pl.ds(c * CH + j * L, L)]
                iv = jnp.clip((tv * scale).astype(jnp.int32), 0, V - 1)
                idx_v[c, pl.ds(j * L, L)] = iv

        def gather(c):
            b = c % NBUF
            return pltpu.async_copy(pe_hbm.at[idx_v.at[c]], rows_v.at[b], gsem[b])

        def put(c):
            b = c % NBUF
            return pltpu.async_copy(
                rows_v.at[b], out_hbm.at[pl.ds(base + c * CH, CH)], osem[b])

        gpend = [None] * NBUF
        opend = [None] * NBUF
        for i in range(NCH + LEAD):
            cg = i
            if cg < NCH:
                b = cg % NBUF
                if opend[b] is not None:
                    opend[b].wait()
                    opend[b] = None
                compute_idx(cg)
                gpend[b] = gather(cg)
            cp = i - LEAD
            if 0 <= cp < NCH:
                b = cp % NBUF
                gpend[b].wait()
                opend[b] = put(cp)
        for p in opend:
            if p is not None:
                p.wait()

    return k


def kernel(t, pe):
    B, = t.shape
    V, D = pe.shape
    return _make_pe_gather(B, V, D)(t, pe)


# speedup vs baseline: 1.5475x; 1.0066x over previous
import functools

import jax
import jax.numpy as jnp
from jax import lax
from jax.experimental import pallas as pl
from jax.experimental.pallas import tpu as pltpu
from jax.experimental.pallas import tpu_sc as plsc


@functools.lru_cache(maxsize=None)
def _make_pe_gather(B, V, D):
    info = plsc.get_sparse_core_info()
    NC, NS, L = info.num_cores, info.num_subcores, info.num_lanes
    NW = NC * NS
    assert B % NW == 0 and D % L == 0
    b_per_w = B // NW
    CH = 64
    NCH = b_per_w // CH
    NBUF = 3
    LEAD = NBUF - 1
    mesh = plsc.VectorSubcoreMesh(core_axis_name="c", subcore_axis_name="s")

    @functools.partial(
        pl.kernel,
        mesh=mesh,
        out_type=jax.ShapeDtypeStruct((B, D), jnp.float32),
        scratch_types=[
            pltpu.VMEM((b_per_w,), jnp.float32),
            pltpu.VMEM((NCH, CH), jnp.int32),
            pltpu.VMEM((NBUF, CH, D), jnp.float32),
        ]
        + [pltpu.SemaphoreType.DMA] * (2 * NBUF),
    )
    def k(t_hbm, pe_hbm, out_hbm, t_v, idx_v, rows_v, *sems):
        gsem = sems[:NBUF]
        osem = sems[NBUF:]
        wid = lax.axis_index("s") * NC + lax.axis_index("c")
        base = wid * b_per_w
        pltpu.sync_copy(t_hbm.at[pl.ds(base, b_per_w)], t_v)
        scale = jnp.float32(V - 1)

        def compute_idx(c):
            for j in range(CH // L):
                tv = t_v[pl.ds(c * CH + j * L, L)]
                iv = jnp.clip((tv * scale).astype(jnp.int32), 0, V - 1)
                idx_v[c, pl.ds(j * L, L)] = iv

        def gather(c):
            b = c % NBUF
            return pltpu.async_copy(pe_hbm.at[idx_v.at[c]], rows_v.at[b], gsem[b])

        def put(c):
            b = c % NBUF
            return pltpu.async_copy(
                rows_v.at[b], out_hbm.at[pl.ds(base + c * CH, CH)], osem[b])

        gpend = [None] * NBUF
        opend = [None] * NBUF
        for i in range(NCH + LEAD):
            cg = i
            if cg < NCH:
                b = cg % NBUF
                if opend[b] is not None:
                    opend[b].wait()
                    opend[b] = None
                compute_idx(cg)
                gpend[b] = gather(cg)
            cp = i - LEAD
            if 0 <= cp < NCH:
                b = cp % NBUF
                gpend[b].wait()
                opend[b] = put(cp)
        for p in opend:
            if p is not None:
                p.wait()

    return k


def kernel(t, pe):
    B, = t.shape
    V, D = pe.shape
    return _make_pe_gather(B, V, D)(t, pe)
